# Initial kernel scaffold; baseline (speedup 1.0000x reference)
import jax
import jax.numpy as jnp
from jax.experimental import pallas as pl


def kernel(x, edge_index, W_enc, b_enc, W_int, W_ext, W_dec, b_dec):
    raise NotImplementedError("write your pallas kernel here")


# speedup vs baseline: 6.0361x; 6.0361x over previous
import functools
import jax
import jax.numpy as jnp
from jax import lax
from jax.experimental import pallas as pl
from jax.experimental.pallas import tpu as pltpu
from jax.experimental.pallas import tpu_sc as plsc

N = 10000
E = 160000
IN = 256
H = 256
OUT = 64
L = 4
STEP = 1.0

NC = 2
NS = 16
NP = 10240
EPT = 10240
NB = EPT // 128
RPT = NP // NS
BR = 2048
GRID = NP // BR

_mesh = plsc.VectorSubcoreMesh(core_axis_name="c", subcore_axis_name="s")


def _deg_body(colp_hbm, ones_hbm, zeros_hbm, deg_hbm, deg_sh, colv, onesv,
              zbuf):
  c = lax.axis_index("c")
  s = lax.axis_index("s")
  q = c * NS + s
  pltpu.sync_copy(colp_hbm.at[s], colv)
  pltpu.sync_copy(ones_hbm, onesv)
  pltpu.sync_copy(zeros_hbm.at[pl.ds(s * RPT, RPT)], zbuf)
  pltpu.sync_copy(zbuf, deg_sh.at[pl.ds(s * RPT, RPT)])
  plsc.subcore_barrier()

  def step(j, carry):
    pltpu.sync_copy(onesv, deg_sh.at[colv.at[j]], add=True)
    return carry
  lax.fori_loop(c * (NB // 2), (c + 1) * (NB // 2), step, 0)
  plsc.subcore_barrier()
  pltpu.sync_copy(deg_sh.at[pl.ds(s * RPT, RPT)], deg_hbm.at[q])


def _deg_call(colp, ones1, zeros1):
  f = pl.kernel(
      _deg_body,
      out_type=jax.ShapeDtypeStruct((NC * NS, RPT), jnp.float32),
      mesh=_mesh,
      scratch_types=[
          pltpu.VMEM_SHARED((NP,), jnp.float32),
          pltpu.VMEM((NB, 128), jnp.int32),
          pltpu.VMEM((128,), jnp.float32),
          pltpu.VMEM((RPT,), jnp.float32),
      ],
  )
  return f(colp, ones1, zeros1)


def _spmm_body(ms_hbm, rowp_hbm, colp_hbm, zeros_hbm, raw_hbm, agg_sh,
               rowv, colv, gbuf, zbuf):
  c = lax.axis_index("c")
  s = lax.axis_index("s")
  q = c * NS + s
  pltpu.sync_copy(rowp_hbm.at[q], rowv)
  pltpu.sync_copy(colp_hbm.at[s], colv)
  pltpu.sync_copy(zeros_hbm, zbuf)
  for t in range(RPT // 64):
    pltpu.sync_copy(zbuf, agg_sh.at[pl.ds(s * RPT + t * 64, 64)])
  plsc.subcore_barrier()

  def step(j, carry):
    pltpu.sync_copy(ms_hbm.at[rowv.at[j]], gbuf)
    pltpu.sync_copy(gbuf, agg_sh.at[colv.at[j]], add=True)
    return carry
  lax.fori_loop(0, NB, step, 0)
  plsc.subcore_barrier()
  pltpu.sync_copy(agg_sh.at[pl.ds(s * RPT, RPT)], raw_hbm.at[q])


def _spmm_call(ms_flat, rowp2, colp, zeros2):
  f = pl.kernel(
      _spmm_body,
      out_type=jax.ShapeDtypeStruct((NC * NS, RPT, 128), jnp.float32),
      mesh=_mesh,
      scratch_types=[
          pltpu.VMEM_SHARED((NP, 128), jnp.float32),
          pltpu.VMEM((NB, 128), jnp.int32),
          pltpu.VMEM((NB, 128), jnp.int32),
          pltpu.VMEM((128, 128), jnp.float32),
          pltpu.VMEM((64, 128), jnp.float32),
      ],
  )
  return f(ms_flat, rowp2, colp, zeros2)


def _sym_body(wi_ref, we_ref, wis_ref, wes_ref):
  r = lax.broadcasted_iota(jnp.int32, (H, H), 0)
  col = lax.broadcasted_iota(jnp.int32, (H, H), 1)
  eye = (r == col).astype(jnp.float32)
  dn = (((1,), (1,)), ((), ()))
  for w_ref, o_ref in ((wi_ref, wis_ref), (we_ref, wes_ref)):
    w = w_ref[...]
    up = jnp.where(col >= r, w, 0.0)
    up1 = jnp.where(col > r, w, 0.0)
    o_ref[...] = up + lax.dot_general(
        eye, up1, dn, preferred_element_type=jnp.float32)


def _sym_call(w_int, w_ext):
  return pl.pallas_call(
      _sym_body,
      out_shape=(jax.ShapeDtypeStruct((H, H), jnp.float32),
                 jax.ShapeDtypeStruct((H, H), jnp.float32)),
  )(w_int, w_ext)


def _dis_of(degp_ref):
  deg = degp_ref[0, :] + degp_ref[1, :] + 1.0
  return lax.rsqrt(deg)[:, None]


_DN_T = (((1,), (1,)), ((), ()))


def _enc_body(x_ref, wenc_ref, benc_ref, wis_ref, degp_ref, h_ref, ms_ref):
  x0 = lax.dot_general(x_ref[...], wenc_ref[...], _DN_T,
                       preferred_element_type=jnp.float32) + benc_ref[...]
  h_ref[...] = x0
  ms = _dis_of(degp_ref) * lax.dot_general(
      x0, wis_ref[...], _DN_T, preferred_element_type=jnp.float32)
  ms_ref[0, :, :] = ms[:, :128]
  ms_ref[1, :, :] = ms[:, 128:]


def _enc_call(xp, w_enc, b_enc2, wi_sym, degp):
  return pl.pallas_call(
      _enc_body,
      grid=(GRID,),
      in_specs=[
          pl.BlockSpec((BR, IN), lambda i: (i, 0)),
          pl.BlockSpec((H, IN), lambda i: (0, 0)),
          pl.BlockSpec((1, H), lambda i: (0, 0)),
          pl.BlockSpec((H, H), lambda i: (0, 0)),
          pl.BlockSpec((2, BR), lambda i: (0, i)),
      ],
      out_specs=[
          pl.BlockSpec((BR, H), lambda i: (i, 0)),
          pl.BlockSpec((2, BR, 128), lambda i: (0, i, 0)),
      ],
      out_shape=(jax.ShapeDtypeStruct((NP, H), jnp.float32),
                 jax.ShapeDtypeStruct((2, NP, 128), jnp.float32)),
  )(xp, w_enc, b_enc2, wi_sym, degp)


def _upd_body(h_ref, ms_ref, raw_ref, degp_ref, wes_ref, wis_ref,
              hn_ref, msn_ref):
  dis = _dis_of(degp_ref)
  raw = jnp.concatenate([raw_ref[0, :, :], raw_ref[1, :, :]], axis=1)
  ms = jnp.concatenate([ms_ref[0, :, :], ms_ref[1, :, :]], axis=1)
  h = h_ref[...]
  agg = dis * (raw + ms)
  hn = jnp.maximum(
      h + STEP * (agg - lax.dot_general(
          h, wes_ref[...], _DN_T, preferred_element_type=jnp.float32)), 0.0)
  hn_ref[...] = hn
  msn = dis * lax.dot_general(hn, wis_ref[...], _DN_T,
                              preferred_element_type=jnp.float32)
  msn_ref[0, :, :] = msn[:, :128]
  msn_ref[1, :, :] = msn[:, 128:]


def _upd_call(h, ms2, raw2, degp, we_sym, wi_sym):
  return pl.pallas_call(
      _upd_body,
      grid=(GRID,),
      in_specs=[
          pl.BlockSpec((BR, H), lambda i: (i, 0)),
          pl.BlockSpec((2, BR, 128), lambda i: (0, i, 0)),
          pl.BlockSpec((2, BR, 128), lambda i: (0, i, 0)),
          pl.BlockSpec((2, BR), lambda i: (0, i)),
          pl.BlockSpec((H, H), lambda i: (0, 0)),
          pl.BlockSpec((H, H), lambda i: (0, 0)),
      ],
      out_specs=[
          pl.BlockSpec((BR, H), lambda i: (i, 0)),
          pl.BlockSpec((2, BR, 128), lambda i: (0, i, 0)),
      ],
      out_shape=(jax.ShapeDtypeStruct((NP, H), jnp.float32),
                 jax.ShapeDtypeStruct((2, NP, 128), jnp.float32)),
  )(h, ms2, raw2, degp, we_sym, wi_sym)


def _last_body(h_ref, ms_ref, raw_ref, degp_ref, wes_ref, wdec_ref, bdec_ref,
               out_ref):
  dis = _dis_of(degp_ref)
  raw = jnp.concatenate([raw_ref[0, :, :], raw_ref[1, :, :]], axis=1)
  ms = jnp.concatenate([ms_ref[0, :, :], ms_ref[1, :, :]], axis=1)
  h = h_ref[...]
  agg = dis * (raw + ms)
  hn = jnp.maximum(
      h + STEP * (agg - lax.dot_general(
          h, wes_ref[...], _DN_T, preferred_element_type=jnp.float32)), 0.0)
  out_ref[...] = lax.dot_general(
      hn, wdec_ref[...], _DN_T,
      preferred_element_type=jnp.float32) + bdec_ref[...]


def _last_call(h, ms2, raw2, degp, we_sym, w_dec, b_dec2):
  return pl.pallas_call(
      _last_body,
      grid=(GRID,),
      in_specs=[
          pl.BlockSpec((BR, H), lambda i: (i, 0)),
          pl.BlockSpec((2, BR, 128), lambda i: (0, i, 0)),
          pl.BlockSpec((2, BR, 128), lambda i: (0, i, 0)),
          pl.BlockSpec((2, BR), lambda i: (0, i)),
          pl.BlockSpec((H, H), lambda i: (0, 0)),
          pl.BlockSpec((OUT, H), lambda i: (0, 0)),
          pl.BlockSpec((1, OUT), lambda i: (0, 0)),
      ],
      out_specs=pl.BlockSpec((BR, OUT), lambda i: (i, 0)),
      out_shape=jax.ShapeDtypeStruct((NP, OUT), jnp.float32),
  )(h, ms2, raw2, degp, we_sym, w_dec, b_dec2)


def kernel(x, edge_index, W_enc, b_enc, W_int, W_ext, W_dec, b_dec):
  xp = jnp.pad(x, ((0, NP - N), (0, 0)))
  row = edge_index[0].astype(jnp.int32)
  col = edge_index[1].astype(jnp.int32)
  pad = jnp.full((NS * EPT - E,), N, jnp.int32)
  rowp = jnp.concatenate([row, pad]).reshape(NS, NB, 128)
  colp = jnp.concatenate([col, pad]).reshape(NS, NB, 128)
  rowp2 = jnp.concatenate([rowp, rowp + NP], axis=0)
  ones1 = jnp.ones((128,), jnp.float32)
  zeros1 = jnp.zeros((NP,), jnp.float32)
  zeros2 = jnp.zeros((64, 128), jnp.float32)
  b_enc2 = b_enc.reshape(1, H)
  b_dec2 = b_dec.reshape(1, OUT)

  wi_sym, we_sym = _sym_call(W_int, W_ext)
  degp = _deg_call(colp, ones1, zeros1).reshape(NC, NP)
  h, ms2 = _enc_call(xp, W_enc, b_enc2, wi_sym, degp)
  for _ in range(L - 1):
    raw2 = _spmm_call(ms2.reshape(NC * NP, 128), rowp2, colp,
                      zeros2).reshape(NC, NP, 128)
    h, ms2 = _upd_call(h, ms2, raw2, degp, we_sym, wi_sym)
  raw2 = _spmm_call(ms2.reshape(NC * NP, 128), rowp2, colp,
                    zeros2).reshape(NC, NP, 128)
  out = _last_call(h, ms2, raw2, degp, we_sym, W_dec, b_dec2)
  return out[:N]


# speedup vs baseline: 7.0216x; 1.1633x over previous
import functools
import jax
import jax.numpy as jnp
from jax import lax
from jax.experimental import pallas as pl
from jax.experimental.pallas import tpu as pltpu
from jax.experimental.pallas import tpu_sc as plsc

N = 10000
E = 160000
IN = 256
H = 256
OUT = 64
L = 4
STEP = 1.0

NC = 2
NS = 16
NP = 10240
EPT = 10240
BLK = 64
NB = EPT // BLK
NST = 4
NBW = NB // NST
RPT = NP // NS
BR = 2048
GRID = NP // BR

_mesh = plsc.VectorSubcoreMesh(core_axis_name="c", subcore_axis_name="s")


def _deg_body(colp_hbm, ones_hbm, zeros_hbm, deg_hbm, deg_sh, colv, onesv,
              zbuf):
  c = lax.axis_index("c")
  s = lax.axis_index("s")
  q = c * NS + s
  pltpu.sync_copy(colp_hbm.at[s], colv)
  pltpu.sync_copy(ones_hbm, onesv)
  pltpu.sync_copy(zeros_hbm.at[pl.ds(s * RPT, RPT)], zbuf)
  pltpu.sync_copy(zbuf, deg_sh.at[pl.ds(s * RPT, RPT)])
  plsc.subcore_barrier()

  def step(j, carry):
    pltpu.sync_copy(onesv, deg_sh.at[colv.at[j]], add=True)
    return carry
  lax.fori_loop(c * (NB // 2), (c + 1) * (NB // 2), step, 0)
  plsc.subcore_barrier()
  pltpu.sync_copy(deg_sh.at[pl.ds(s * RPT, RPT)], deg_hbm.at[q])


def _deg_call(colp, ones1, zeros1):
  f = pl.kernel(
      _deg_body,
      out_type=jax.ShapeDtypeStruct((NC * NS, RPT), jnp.float32),
      mesh=_mesh,
      scratch_types=[
          pltpu.VMEM_SHARED((NP,), jnp.float32),
          pltpu.VMEM((NB, BLK), jnp.int32),
          pltpu.VMEM((BLK,), jnp.float32),
          pltpu.VMEM((RPT,), jnp.float32),
      ],
  )
  return f(colp, ones1, zeros1)


def _spmm_body(ms_hbm, rowp_hbm, colp_hbm, zeros_hbm, raw_hbm, agg_sh,
               rowv, colv, gbuf0, gbuf1, gsem0, gsem1):
  c = lax.axis_index("c")
  s = lax.axis_index("s")
  q = c * NS + s
  pltpu.sync_copy(zeros_hbm, gbuf0)
  for t in range(RPT // BLK):
    pltpu.sync_copy(gbuf0, agg_sh.at[pl.ds(s * RPT + t * BLK, BLK)])
  plsc.subcore_barrier()

  for t in range(NST):
    pltpu.sync_copy(rowp_hbm.at[q * NST + t], rowv)
    pltpu.sync_copy(colp_hbm.at[s * NST + t], colv)
    pltpu.async_copy(ms_hbm.at[rowv.at[0]], gbuf0, gsem0)
    pltpu.async_copy(ms_hbm.at[rowv.at[1]], gbuf1, gsem1)

    def outer(k, carry):
      j0 = 2 * k
      for b, (gb, gs) in enumerate(((gbuf0, gsem0), (gbuf1, gsem1))):
        j = j0 + b
        pltpu.make_async_copy(ms_hbm.at[rowv.at[j]], gb, gs).wait()
        pltpu.sync_copy(gb, agg_sh.at[colv.at[j]], add=True)
        jn = jnp.minimum(j + 2, NBW - 1)
        pltpu.async_copy(ms_hbm.at[rowv.at[jn]], gb, gs)
      return carry

    lax.fori_loop(0, NBW // 2, outer, 0)
    pltpu.make_async_copy(ms_hbm.at[rowv.at[0]], gbuf0, gsem0).wait()
    pltpu.make_async_copy(ms_hbm.at[rowv.at[0]], gbuf1, gsem1).wait()
  plsc.subcore_barrier()
  pltpu.sync_copy(agg_sh.at[pl.ds(s * RPT, RPT)], raw_hbm.at[q])


def _spmm_call(ms_flat, rowp2, colp, zeros2):
  f = pl.kernel(
      _spmm_body,
      out_type=jax.ShapeDtypeStruct((NC * NS, RPT, 128), jnp.float32),
      mesh=_mesh,
      scratch_types=[
          pltpu.VMEM_SHARED((NP, 128), jnp.float32),
          pltpu.VMEM((NBW, BLK), jnp.int32),
          pltpu.VMEM((NBW, BLK), jnp.int32),
          pltpu.VMEM((BLK, 128), jnp.float32),
          pltpu.VMEM((BLK, 128), jnp.float32),
          pltpu.SemaphoreType.DMA,
          pltpu.SemaphoreType.DMA,
      ],
  )
  return f(ms_flat, rowp2, colp, zeros2)


def _sym_body(wi_ref, we_ref, wis_ref, wes_ref):
  r = lax.broadcasted_iota(jnp.int32, (H, H), 0)
  col = lax.broadcasted_iota(jnp.int32, (H, H), 1)
  eye = (r == col).astype(jnp.float32)
  dn = (((1,), (1,)), ((), ()))
  for w_ref, o_ref in ((wi_ref, wis_ref), (we_ref, wes_ref)):
    w = w_ref[...]
    up = jnp.where(col >= r, w, 0.0)
    up1 = jnp.where(col > r, w, 0.0)
    o_ref[...] = up + lax.dot_general(
        eye, up1, dn, preferred_element_type=jnp.float32)


def _sym_call(w_int, w_ext):
  return pl.pallas_call(
      _sym_body,
      out_shape=(jax.ShapeDtypeStruct((H, H), jnp.float32),
                 jax.ShapeDtypeStruct((H, H), jnp.float32)),
  )(w_int, w_ext)


def _dis_of(degp_ref):
  deg = degp_ref[0, :] + degp_ref[1, :] + 1.0
  return lax.rsqrt(deg)[:, None]


_DN_T = (((1,), (1,)), ((), ()))


def _enc_body(x_ref, wenc_ref, benc_ref, wis_ref, degp_ref, h_ref, ms_ref):
  x0 = lax.dot_general(x_ref[...], wenc_ref[...], _DN_T,
                       preferred_element_type=jnp.float32) + benc_ref[...]
  h_ref[...] = x0
  ms = _dis_of(degp_ref) * lax.dot_general(
      x0, wis_ref[...], _DN_T, preferred_element_type=jnp.float32)
  ms_ref[0, :, :] = ms[:, :128]
  ms_ref[1, :, :] = ms[:, 128:]


def _enc_call(xp, w_enc, b_enc2, wi_sym, degp):
  return pl.pallas_call(
      _enc_body,
      grid=(GRID,),
      in_specs=[
          pl.BlockSpec((BR, IN), lambda i: (i, 0)),
          pl.BlockSpec((H, IN), lambda i: (0, 0)),
          pl.BlockSpec((1, H), lambda i: (0, 0)),
          pl.BlockSpec((H, H), lambda i: (0, 0)),
          pl.BlockSpec((2, BR), lambda i: (0, i)),
      ],
      out_specs=[
          pl.BlockSpec((BR, H), lambda i: (i, 0)),
          pl.BlockSpec((2, BR, 128), lambda i: (0, i, 0)),
      ],
      out_shape=(jax.ShapeDtypeStruct((NP, H), jnp.float32),
                 jax.ShapeDtypeStruct((2, NP, 128), jnp.float32)),
  )(xp, w_enc, b_enc2, wi_sym, degp)


def _upd_body(h_ref, ms_ref, raw_ref, degp_ref, wes_ref, wis_ref,
              hn_ref, msn_ref):
  dis = _dis_of(degp_ref)
  raw = jnp.concatenate([raw_ref[0, :, :], raw_ref[1, :, :]], axis=1)
  ms = jnp.concatenate([ms_ref[0, :, :], ms_ref[1, :, :]], axis=1)
  h = h_ref[...]
  agg = dis * (raw + ms)
  hn = jnp.maximum(
      h + STEP * (agg - lax.dot_general(
          h, wes_ref[...], _DN_T, preferred_element_type=jnp.float32)), 0.0)
  hn_ref[...] = hn
  msn = dis * lax.dot_general(hn, wis_ref[...], _DN_T,
                              preferred_element_type=jnp.float32)
  msn_ref[0, :, :] = msn[:, :128]
  msn_ref[1, :, :] = msn[:, 128:]


def _upd_call(h, ms2, raw2, degp, we_sym, wi_sym):
  return pl.pallas_call(
      _upd_body,
      grid=(GRID,),
      in_specs=[
          pl.BlockSpec((BR, H), lambda i: (i, 0)),
          pl.BlockSpec((2, BR, 128), lambda i: (0, i, 0)),
          pl.BlockSpec((2, BR, 128), lambda i: (0, i, 0)),
          pl.BlockSpec((2, BR), lambda i: (0, i)),
          pl.BlockSpec((H, H), lambda i: (0, 0)),
          pl.BlockSpec((H, H), lambda i: (0, 0)),
      ],
      out_specs=[
          pl.BlockSpec((BR, H), lambda i: (i, 0)),
          pl.BlockSpec((2, BR, 128), lambda i: (0, i, 0)),
      ],
      out_shape=(jax.ShapeDtypeStruct((NP, H), jnp.float32),
                 jax.ShapeDtypeStruct((2, NP, 128), jnp.float32)),
  )(h, ms2, raw2, degp, we_sym, wi_sym)


def _last_body(h_ref, ms_ref, raw_ref, degp_ref, wes_ref, wdec_ref, bdec_ref,
               out_ref):
  dis = _dis_of(degp_ref)
  raw = jnp.concatenate([raw_ref[0, :, :], raw_ref[1, :, :]], axis=1)
  ms = jnp.concatenate([ms_ref[0, :, :], ms_ref[1, :, :]], axis=1)
  h = h_ref[...]
  agg = dis * (raw + ms)
  hn = jnp.maximum(
      h + STEP * (agg - lax.dot_general(
          h, wes_ref[...], _DN_T, preferred_element_type=jnp.float32)), 0.0)
  out_ref[...] = lax.dot_general(
      hn, wdec_ref[...], _DN_T,
      preferred_element_type=jnp.float32) + bdec_ref[...]


def _last_call(h, ms2, raw2, degp, we_sym, w_dec, b_dec2):
  return pl.pallas_call(
      _last_body,
      grid=(GRID,),
      in_specs=[
          pl.BlockSpec((BR, H), lambda i: (i, 0)),
          pl.BlockSpec((2, BR, 128), lambda i: (0, i, 0)),
          pl.BlockSpec((2, BR, 128), lambda i: (0, i, 0)),
          pl.BlockSpec((2, BR), lambda i: (0, i)),
          pl.BlockSpec((H, H), lambda i: (0, 0)),
          pl.BlockSpec((OUT, H), lambda i: (0, 0)),
          pl.BlockSpec((1, OUT), lambda i: (0, 0)),
      ],
      out_specs=pl.BlockSpec((BR, OUT), lambda i: (i, 0)),
      out_shape=jax.ShapeDtypeStruct((NP, OUT), jnp.float32),
  )(h, ms2, raw2, degp, we_sym, w_dec, b_dec2)


def kernel(x, edge_index, W_enc, b_enc, W_int, W_ext, W_dec, b_dec):
  xp = jnp.pad(x, ((0, NP - N), (0, 0)))
  row = edge_index[0].astype(jnp.int32)
  col = edge_index[1].astype(jnp.int32)
  pad = jnp.full((NS * EPT - E,), N, jnp.int32)
  rowp = jnp.concatenate([row, pad]).reshape(NS, NB, BLK)
  colp = jnp.concatenate([col, pad]).reshape(NS, NB, BLK)
  rowp2 = jnp.concatenate([rowp, rowp + NP],
                          axis=0).reshape(NC * NS * NST, NBW, BLK)
  colpw = colp.reshape(NS * NST, NBW, BLK)
  ones1 = jnp.ones((BLK,), jnp.float32)
  zeros1 = jnp.zeros((NP,), jnp.float32)
  zeros2 = jnp.zeros((BLK, 128), jnp.float32)
  b_enc2 = b_enc.reshape(1, H)
  b_dec2 = b_dec.reshape(1, OUT)

  wi_sym, we_sym = _sym_call(W_int, W_ext)
  degp = _deg_call(colp, ones1, zeros1).reshape(NC, NP)
  h, ms2 = _enc_call(xp, W_enc, b_enc2, wi_sym, degp)
  for _ in range(L - 1):
    raw2 = _spmm_call(ms2.reshape(NC * NP, 128), rowp2, colpw,
                      zeros2).reshape(NC, NP, 128)
    h, ms2 = _upd_call(h, ms2, raw2, degp, we_sym, wi_sym)
  raw2 = _spmm_call(ms2.reshape(NC * NP, 128), rowp2, colpw,
                    zeros2).reshape(NC, NP, 128)
  out = _last_call(h, ms2, raw2, degp, we_sym, W_dec, b_dec2)
  return out[:N]
